# Initial kernel scaffold; baseline (speedup 1.0000x reference)
import jax
import jax.numpy as jnp
from jax.experimental import pallas as pl


def kernel(x, edge_index, W1, b1, g1, be1, W2, b2, g2, be2, Wf, bf):
    raise NotImplementedError("write your pallas kernel here")


# speedup vs baseline: 17.9278x; 17.9278x over previous
import jax
import jax.numpy as jnp
from jax import lax
from jax.experimental import pallas as pl
from jax.experimental.pallas import tpu as pltpu
from jax.experimental.pallas import tpu_sc as plsc

N = 10000
E = 320000
D = 128
H = 128

NC = 2
NS = 16
NW = NC * NS
C = 80
NCHUNK = 126
EP = NW * NCHUNK * C
NP = 10112
ZR = NP // NS
NBUF = 2
RB = 1000


def _sc_hist_body(d3_hbm, ones_hbm, zdeg_hbm, out_hbm, didx, ones_v, acc):
    cid = lax.axis_index("c")
    sid = lax.axis_index("s")
    wid = cid * NS + sid
    pltpu.sync_copy(d3_hbm.at[wid], didx)
    pltpu.sync_copy(ones_hbm, ones_v)
    row0 = pl.multiple_of(sid * ZR, 8)
    pltpu.sync_copy(zdeg_hbm, acc.at[pl.ds(row0, ZR)])
    plsc.subcore_barrier()

    def body(j, carry):
        pltpu.sync_copy(ones_v, acc.at[didx.at[j]], add=True)
        return carry

    lax.fori_loop(0, NCHUNK, body, 0)
    plsc.subcore_barrier()
    pltpu.sync_copy(acc.at[pl.ds(row0, ZR)],
                    out_hbm.at[cid, pl.ds(row0, ZR)])


def _sc_hist(d3, ones, zdeg):
    mesh = plsc.VectorSubcoreMesh(core_axis_name="c", subcore_axis_name="s")
    return pl.kernel(
        _sc_hist_body,
        out_type=jax.ShapeDtypeStruct((NC, NP, 1), jnp.float32),
        mesh=mesh,
        scratch_types=[
            pltpu.VMEM((NCHUNK, C), jnp.int32),
            pltpu.VMEM((C, 1), jnp.float32),
            pltpu.VMEM_SHARED((NP, 1), jnp.float32),
        ],
    )(d3, ones, zdeg)


def _sc_agg_body(s1_hbm, d3_hbm, hl_hbm, zrows_hbm, out_hbm,
                 sidx, didx, buf0, buf1, acc, sem0, sem1):
    cid = lax.axis_index("c")
    sid = lax.axis_index("s")
    wid = cid * NS + sid
    pltpu.sync_copy(s1_hbm.at[wid], sidx)
    pltpu.sync_copy(d3_hbm.at[wid], didx)
    row0 = pl.multiple_of(sid * ZR, 8)
    pltpu.sync_copy(zrows_hbm, acc.at[pl.ds(row0, ZR)])
    plsc.subcore_barrier()

    bufs = (buf0, buf1)
    sems = (sem0, sem1)

    def _sidx(j):
        return sidx.at[pl.ds(pl.multiple_of(j * C, 8), C)]

    for b in range(NBUF):
        pltpu.async_copy(hl_hbm.at[_sidx(b)], bufs[b], sems[b])

    def body(i, carry):
        for b in range(NBUF):
            j = i * NBUF + b
            pltpu.make_async_copy(hl_hbm.at[_sidx(j)], bufs[b],
                                  sems[b]).wait()
            pltpu.sync_copy(bufs[b], acc.at[didx.at[j]], add=True)
            pltpu.async_copy(hl_hbm.at[_sidx(j + NBUF)], bufs[b], sems[b])
        return carry

    lax.fori_loop(0, NCHUNK // NBUF - 1, body, 0)
    for b in range(NBUF):
        j = NCHUNK - NBUF + b
        pltpu.make_async_copy(hl_hbm.at[_sidx(j)], bufs[b], sems[b]).wait()
        pltpu.sync_copy(bufs[b], acc.at[didx.at[j]], add=True)

    plsc.subcore_barrier()
    pltpu.sync_copy(acc.at[pl.ds(row0, ZR)],
                    out_hbm.at[cid, pl.ds(row0, ZR)])


def _sc_agg(s1, d3, hl, zrows):
    mesh = plsc.VectorSubcoreMesh(core_axis_name="c", subcore_axis_name="s")
    return pl.kernel(
        _sc_agg_body,
        out_type=jax.ShapeDtypeStruct((NC, NP, H), jnp.float32),
        mesh=mesh,
        scratch_types=[
            pltpu.VMEM((NCHUNK * C,), jnp.int32),
            pltpu.VMEM((NCHUNK, C), jnp.int32),
            pltpu.VMEM((C, H), jnp.float32),
            pltpu.VMEM((C, H), jnp.float32),
            pltpu.VMEM_SHARED((NP, H), jnp.float32),
            pltpu.SemaphoreType.DMA,
            pltpu.SemaphoreType.DMA,
        ],
    )(s1, d3, hl, zrows)


def _tc_l1_body(degp_ref, x_ref, w_ref, hlp_ref, dis_ref):
    deg = degp_ref[0] + degp_ref[1] + 1.0
    dis = lax.rsqrt(deg)
    hl = jnp.dot(x_ref[...], w_ref[...], preferred_element_type=jnp.float32)
    hlp_ref[...] = hl * dis
    dis_ref[...] = dis


def _tc_l1(degp, x, W1):
    return pl.pallas_call(
        _tc_l1_body,
        grid=(N // RB,),
        in_specs=[
            pl.BlockSpec((NC, RB, 1), lambda i: (0, i, 0)),
            pl.BlockSpec((RB, D), lambda i: (i, 0)),
            pl.BlockSpec((D, H), lambda i: (0, 0)),
        ],
        out_specs=[
            pl.BlockSpec((RB, H), lambda i: (i, 0)),
            pl.BlockSpec((RB, 1), lambda i: (i, 0)),
        ],
        out_shape=[
            jax.ShapeDtypeStruct((N, H), jnp.float32),
            jax.ShapeDtypeStruct((N, 1), jnp.float32),
        ],
    )(degp, x, W1)


def _bn_relu(t, g, be):
    mu = jnp.mean(t, axis=0, keepdims=True)
    xc = t - mu
    var = jnp.mean(xc * xc, axis=0, keepdims=True)
    return jnp.maximum(xc * lax.rsqrt(var + 1e-5) * g + be, 0.0)


def _tc_mid_body(p_ref, hlp_ref, dis_ref, b_ref, g_ref, be_ref, w_ref,
                 out_ref):
    agg = p_ref[0, :N] + p_ref[1, :N] + hlp_ref[...]
    t = agg * dis_ref[...] + b_ref[...]
    h2 = _bn_relu(t, g_ref[...], be_ref[...])
    out_ref[...] = jnp.dot(h2, w_ref[...],
                           preferred_element_type=jnp.float32) * dis_ref[...]


def _tc_mid(p, hlp, dis, b, g, be, W):
    return pl.pallas_call(
        _tc_mid_body,
        out_shape=jax.ShapeDtypeStruct((N, H), jnp.float32),
    )(p, hlp, dis, b, g, be, W)


def _tc_final_body(p_ref, hlp_ref, dis_ref, b_ref, g_ref, be_ref, w_ref,
                   bf_ref, out_ref):
    agg = p_ref[0, :N] + p_ref[1, :N] + hlp_ref[...]
    t = agg * dis_ref[...] + b_ref[...]
    h3 = _bn_relu(t, g_ref[...], be_ref[...])
    out_ref[...] = jnp.dot(h3, w_ref[...],
                           preferred_element_type=jnp.float32) + bf_ref[...]


def _tc_final(p, hlp, dis, b, g, be, Wf, bf):
    return pl.pallas_call(
        _tc_final_body,
        out_shape=jax.ShapeDtypeStruct((N, H), jnp.float32),
    )(p, hlp, dis, b, g, be, Wf, bf)


def kernel(x, edge_index, W1, b1, g1, be1, W2, b2, g2, be2, Wf, bf):
    pad = EP - E
    s_pad = jnp.concatenate(
        [edge_index[0], jnp.zeros((pad,), jnp.int32)])
    d_pad = jnp.concatenate(
        [edge_index[1], jnp.full((pad,), N, jnp.int32)])
    s1 = s_pad.reshape(NW, NCHUNK * C)
    d3 = d_pad.reshape(NW, NCHUNK, C)
    ones = jnp.ones((C, 1), jnp.float32)
    zdeg = jnp.zeros((ZR, 1), jnp.float32)
    zrows = jnp.zeros((ZR, H), jnp.float32)
    b1r, g1r, be1r = b1.reshape(1, H), g1.reshape(1, H), be1.reshape(1, H)
    b2r, g2r, be2r = b2.reshape(1, H), g2.reshape(1, H), be2.reshape(1, H)
    bfr = bf.reshape(1, H)

    degp = _sc_hist(d3, ones, zdeg)
    hlp1, dis = _tc_l1(degp, x, W1)
    p1 = _sc_agg(s1, d3, hlp1, zrows)
    hlp2 = _tc_mid(p1, hlp1, dis, b1r, g1r, be1r, W2)
    p2 = _sc_agg(s1, d3, hlp2, zrows)
    return _tc_final(p2, hlp2, dis, b2r, g2r, be2r, Wf, bfr)
